# baseline (speedup 1.0000x reference)
import jax
import jax.numpy as jnp
from jax import lax
from jax.experimental import pallas as pl
from jax.experimental.pallas import tpu as pltpu
from jax.experimental.pallas import tpu_sc as plsc

B = 16384
D = 16
NC = 2
NS = 16
L = 16
NW = NC * NS
BPW = B // NW
NBLK = BPW // L


def _mf_body(uid_table, iid_table, uidx_hbm, iidx_hbm, out_hbm,
             uidx_v, iidx_v, u_rows, i_rows, out_v, sem):
    wid = lax.axis_index("s") * NC + lax.axis_index("c")
    base = wid * BPW
    pltpu.sync_copy(uidx_hbm.at[pl.ds(base, BPW)], uidx_v)
    pltpu.sync_copy(iidx_hbm.at[pl.ds(base, BPW)], iidx_v)
    cu = pltpu.async_copy(uid_table.at[uidx_v], u_rows, sem)
    ci = pltpu.async_copy(iid_table.at[iidx_v], i_rows, sem)
    cu.wait()
    ci.wait()

    lanes = lax.iota(jnp.int32, L)

    def blk_body(blk, carry):
        row0 = blk * L
        rows = row0 + lanes
        acc = jnp.zeros((L,), jnp.float32)
        for d in range(D):
            cols = jnp.full((L,), d, jnp.int32)
            u = plsc.load_gather(u_rows, [rows, cols])
            v = plsc.load_gather(i_rows, [rows, cols])
            acc = acc + u * v
        out_v[pl.ds(row0, L)] = acc
        return carry

    lax.fori_loop(0, NBLK, blk_body, 0)
    pltpu.sync_copy(out_v, out_hbm.at[pl.ds(base, BPW)])


@jax.jit
def kernel(x, uid_table, iid_table):
    uidx = x[:, 0]
    iidx = x[:, 1]
    k = pl.kernel(
        _mf_body,
        out_type=jax.ShapeDtypeStruct((B,), jnp.float32),
        mesh=plsc.VectorSubcoreMesh(core_axis_name="c", subcore_axis_name="s"),
        scratch_types=[
            pltpu.VMEM((BPW,), jnp.int32),
            pltpu.VMEM((BPW,), jnp.int32),
            pltpu.VMEM((BPW, D), jnp.float32),
            pltpu.VMEM((BPW, D), jnp.float32),
            pltpu.VMEM((BPW,), jnp.float32),
            pltpu.SemaphoreType.DMA,
        ],
        compiler_params=pltpu.CompilerParams(
            use_tc_tiling_on_sc=False, needs_layout_passes=False
        ),
    )
    return k(uid_table, iid_table, uidx, iidx)
